# baseline (speedup 1.0000x reference)
import jax
import jax.numpy as jnp
from jax.experimental import pallas as pl
from jax.experimental.pallas import tpu as pltpu

B = 4096
D = 2048
K = 1000
BR = 512


def _fused_kernel(x_ref, w_ref, b_ref, g_ref, idx_ref, ent_ref):
    logits = jax.lax.dot_general(
        x_ref[...], w_ref[...],
        dimension_numbers=(((1,), (1,)), ((), ())),
        preferred_element_type=jnp.float32) + b_ref[...]

    m = jnp.max(logits, axis=1, keepdims=True)
    e = jnp.exp(logits - m)
    s = jnp.sum(e, axis=1, keepdims=True)
    p2 = e / s + jnp.float32(1e-8)
    ent_ref[...] = jnp.sum(-p2 * jnp.log(p2), axis=1, keepdims=True)

    z = logits + g_ref[...]
    idx_ref[...] = jnp.argmax(z, axis=1).astype(jnp.int32)[:, None]


@jax.jit
def kernel(x, W, b):
    g = jax.random.gumbel(jax.random.key(42), (B, K), jnp.float32)
    bp = b.reshape(1, K)

    grid = (B // BR,)
    idx, ent = pl.pallas_call(
        _fused_kernel,
        grid=grid,
        in_specs=[
            pl.BlockSpec((BR, D), lambda i: (i, 0)),
            pl.BlockSpec((K, D), lambda i: (0, 0)),
            pl.BlockSpec((1, K), lambda i: (0, 0)),
            pl.BlockSpec((BR, K), lambda i: (i, 0)),
        ],
        out_specs=[
            pl.BlockSpec((BR, 1), lambda i: (i, 0)),
            pl.BlockSpec((BR, 1), lambda i: (i, 0)),
        ],
        out_shape=[
            jax.ShapeDtypeStruct((B, 1), jnp.int32),
            jax.ShapeDtypeStruct((B, 1), jnp.float32),
        ],
        compiler_params=pltpu.CompilerParams(
            dimension_semantics=("parallel",),
        ),
    )(x, W, bp, g)
    return idx, jnp.sum(ent)


# speedup vs baseline: 1.0584x; 1.0584x over previous
import jax
import jax.numpy as jnp
from jax.experimental import pallas as pl
from jax.experimental.pallas import tpu as pltpu

B = 4096
D = 2048
K = 1000
BR = 512

_KS0 = 0
_KS1 = 42
_KS2 = 0x1BD11BDA ^ 42
_ROT_A = (13, 15, 26, 6)
_ROT_B = (17, 29, 16, 24)
_TINY = 1.1754943508222875e-38


def _rotl(x, d):
    return (x << jnp.uint32(d)) | (x >> jnp.uint32(32 - d))


def _four_rounds(x0, x1, rots):
    for r in rots:
        x0 = x0 + x1
        x1 = _rotl(x1, r) ^ x0
    return x0, x1


def _gumbel_block(base):
    row = jax.lax.broadcasted_iota(jnp.uint32, (BR, K), 0)
    col = jax.lax.broadcasted_iota(jnp.uint32, (BR, K), 1)
    cnt = base + row * jnp.uint32(K) + col
    x0 = jnp.full((BR, K), jnp.uint32(_KS0))
    x1 = cnt + jnp.uint32(_KS1)
    x0, x1 = _four_rounds(x0, x1, _ROT_A)
    x0, x1 = x0 + jnp.uint32(_KS1), x1 + jnp.uint32(_KS2 + 1)
    x0, x1 = _four_rounds(x0, x1, _ROT_B)
    x0, x1 = x0 + jnp.uint32(_KS2), x1 + jnp.uint32(_KS0 + 2)
    x0, x1 = _four_rounds(x0, x1, _ROT_A)
    x0, x1 = x0 + jnp.uint32(_KS0), x1 + jnp.uint32(_KS1 + 3)
    x0, x1 = _four_rounds(x0, x1, _ROT_B)
    x0, x1 = x0 + jnp.uint32(_KS1), x1 + jnp.uint32(_KS2 + 4)
    x0, x1 = _four_rounds(x0, x1, _ROT_A)
    x0, x1 = x0 + jnp.uint32(_KS2), x1 + jnp.uint32(_KS0 + 5)
    bits = x0 ^ x1
    fb = (bits >> jnp.uint32(9)) | jnp.uint32(0x3F800000)
    f = jax.lax.bitcast_convert_type(fb, jnp.float32) - jnp.float32(1.0)
    u = jnp.maximum(jnp.float32(_TINY), f + jnp.float32(_TINY))
    return -jnp.log(-jnp.log(u))


def _fused_kernel(x_ref, w_ref, b_ref, idx_ref, ent_ref):
    logits = jax.lax.dot_general(
        x_ref[...], w_ref[...],
        dimension_numbers=(((1,), (1,)), ((), ())),
        preferred_element_type=jnp.float32) + b_ref[...]

    base = jnp.uint32(pl.program_id(0) * (BR * K))
    z = logits + _gumbel_block(base)
    idx_ref[...] = jnp.argmax(z, axis=1).astype(jnp.int32)[:, None]

    m = jnp.max(logits, axis=1, keepdims=True)
    e = jnp.exp(logits - m)
    s = jnp.sum(e, axis=1, keepdims=True)
    p2 = e / s + jnp.float32(1e-8)
    ent_ref[...] = jnp.sum(-p2 * jnp.log(p2), axis=1, keepdims=True)


@jax.jit
def kernel(x, W, b):
    bp = b.reshape(1, K)
    grid = (B // BR,)
    idx, ent = pl.pallas_call(
        _fused_kernel,
        grid=grid,
        in_specs=[
            pl.BlockSpec((BR, D), lambda i: (i, 0)),
            pl.BlockSpec((K, D), lambda i: (0, 0)),
            pl.BlockSpec((1, K), lambda i: (0, 0)),
        ],
        out_specs=[
            pl.BlockSpec((BR, 1), lambda i: (i, 0)),
            pl.BlockSpec((BR, 1), lambda i: (i, 0)),
        ],
        out_shape=[
            jax.ShapeDtypeStruct((B, 1), jnp.int32),
            jax.ShapeDtypeStruct((B, 1), jnp.float32),
        ],
        compiler_params=pltpu.CompilerParams(
            dimension_semantics=("parallel",),
        ),
    )(x, W, bp)
    return idx, jnp.sum(ent)


# speedup vs baseline: 1.0633x; 1.0047x over previous
import jax
import jax.numpy as jnp
from jax.experimental import pallas as pl
from jax.experimental.pallas import tpu as pltpu

B = 4096
D = 2048
K = 1000
BR = 256

_KS0 = 0
_KS1 = 42
_KS2 = 0x1BD11BDA ^ 42
_ROT_A = (13, 15, 26, 6)
_ROT_B = (17, 29, 16, 24)
_TINY = 1.1754943508222875e-38


def _rotl(x, d):
    return (x << jnp.uint32(d)) | (x >> jnp.uint32(32 - d))


def _four_rounds(x0, x1, rots):
    for r in rots:
        x0 = x0 + x1
        x1 = _rotl(x1, r) ^ x0
    return x0, x1


def _gumbel_block(base):
    row = jax.lax.broadcasted_iota(jnp.uint32, (BR, K), 0)
    col = jax.lax.broadcasted_iota(jnp.uint32, (BR, K), 1)
    cnt = base + row * jnp.uint32(K) + col
    x0 = jnp.full((BR, K), jnp.uint32(_KS0))
    x1 = cnt + jnp.uint32(_KS1)
    x0, x1 = _four_rounds(x0, x1, _ROT_A)
    x0, x1 = x0 + jnp.uint32(_KS1), x1 + jnp.uint32(_KS2 + 1)
    x0, x1 = _four_rounds(x0, x1, _ROT_B)
    x0, x1 = x0 + jnp.uint32(_KS2), x1 + jnp.uint32(_KS0 + 2)
    x0, x1 = _four_rounds(x0, x1, _ROT_A)
    x0, x1 = x0 + jnp.uint32(_KS0), x1 + jnp.uint32(_KS1 + 3)
    x0, x1 = _four_rounds(x0, x1, _ROT_B)
    x0, x1 = x0 + jnp.uint32(_KS1), x1 + jnp.uint32(_KS2 + 4)
    x0, x1 = _four_rounds(x0, x1, _ROT_A)
    x0, x1 = x0 + jnp.uint32(_KS2), x1 + jnp.uint32(_KS0 + 5)
    bits = x0 ^ x1
    fb = (bits >> jnp.uint32(9)) | jnp.uint32(0x3F800000)
    f = jax.lax.bitcast_convert_type(fb, jnp.float32) - jnp.float32(1.0)
    u = jnp.maximum(jnp.float32(_TINY), f + jnp.float32(_TINY))
    return -jnp.log(-jnp.log(u))


def _fused_kernel(x_ref, w_ref, b_ref, idx_ref, ent_ref):
    logits = jax.lax.dot_general(
        x_ref[...], w_ref[...],
        dimension_numbers=(((1,), (1,)), ((), ())),
        preferred_element_type=jnp.float32) + b_ref[...]

    base = jnp.uint32(pl.program_id(0) * (BR * K))
    z = logits + _gumbel_block(base)
    idx_ref[...] = jnp.argmax(z, axis=1).astype(jnp.int32)[:, None]

    m = jnp.max(logits, axis=1, keepdims=True)
    e = jnp.exp(logits - m)
    s = jnp.sum(e, axis=1, keepdims=True)
    p2 = e / s + jnp.float32(1e-8)
    ent_ref[...] = jnp.sum(-p2 * jnp.log(p2), axis=1, keepdims=True)


@jax.jit
def kernel(x, W, b):
    bp = b.reshape(1, K)
    grid = (B // BR,)
    idx, ent = pl.pallas_call(
        _fused_kernel,
        grid=grid,
        in_specs=[
            pl.BlockSpec((BR, D), lambda i: (i, 0)),
            pl.BlockSpec((K, D), lambda i: (0, 0)),
            pl.BlockSpec((1, K), lambda i: (0, 0)),
        ],
        out_specs=[
            pl.BlockSpec((BR, 1), lambda i: (i, 0)),
            pl.BlockSpec((BR, 1), lambda i: (i, 0)),
        ],
        out_shape=[
            jax.ShapeDtypeStruct((B, 1), jnp.int32),
            jax.ShapeDtypeStruct((B, 1), jnp.float32),
        ],
        compiler_params=pltpu.CompilerParams(
            dimension_semantics=("parallel",),
        ),
    )(x, W, bp)
    return idx, jnp.sum(ent)


# speedup vs baseline: 1.1366x; 1.0689x over previous
import jax
import jax.numpy as jnp
from jax.experimental import pallas as pl
from jax.experimental.pallas import tpu as pltpu

B = 4096
D = 2048
K = 1000
BR = 1024

_KS0 = 0
_KS1 = 42
_KS2 = 0x1BD11BDA ^ 42
_ROT_A = (13, 15, 26, 6)
_ROT_B = (17, 29, 16, 24)
_TINY = 1.1754943508222875e-38


def _rotl(x, d):
    return (x << jnp.uint32(d)) | (x >> jnp.uint32(32 - d))


def _four_rounds(x0, x1, rots):
    for r in rots:
        x0 = x0 + x1
        x1 = _rotl(x1, r) ^ x0
    return x0, x1


def _gumbel_block(base):
    row = jax.lax.broadcasted_iota(jnp.uint32, (BR, K), 0)
    col = jax.lax.broadcasted_iota(jnp.uint32, (BR, K), 1)
    cnt = base + row * jnp.uint32(K) + col
    x0 = jnp.full((BR, K), jnp.uint32(_KS0))
    x1 = cnt + jnp.uint32(_KS1)
    x0, x1 = _four_rounds(x0, x1, _ROT_A)
    x0, x1 = x0 + jnp.uint32(_KS1), x1 + jnp.uint32(_KS2 + 1)
    x0, x1 = _four_rounds(x0, x1, _ROT_B)
    x0, x1 = x0 + jnp.uint32(_KS2), x1 + jnp.uint32(_KS0 + 2)
    x0, x1 = _four_rounds(x0, x1, _ROT_A)
    x0, x1 = x0 + jnp.uint32(_KS0), x1 + jnp.uint32(_KS1 + 3)
    x0, x1 = _four_rounds(x0, x1, _ROT_B)
    x0, x1 = x0 + jnp.uint32(_KS1), x1 + jnp.uint32(_KS2 + 4)
    x0, x1 = _four_rounds(x0, x1, _ROT_A)
    x0, x1 = x0 + jnp.uint32(_KS2), x1 + jnp.uint32(_KS0 + 5)
    bits = x0 ^ x1
    fb = (bits >> jnp.uint32(9)) | jnp.uint32(0x3F800000)
    f = jax.lax.bitcast_convert_type(fb, jnp.float32) - jnp.float32(1.0)
    u = jnp.maximum(jnp.float32(_TINY), f + jnp.float32(_TINY))
    return -jnp.log(-jnp.log(u))


def _fused_kernel(x_ref, w_ref, b_ref, idx_ref, ent_ref):
    logits = jax.lax.dot_general(
        x_ref[...], w_ref[...],
        dimension_numbers=(((1,), (1,)), ((), ())),
        preferred_element_type=jnp.float32) + b_ref[...]

    base = jnp.uint32(pl.program_id(0) * (BR * K))
    z = logits + _gumbel_block(base)
    idx_ref[...] = jnp.argmax(z, axis=1).astype(jnp.int32)[:, None]

    m = jnp.max(logits, axis=1, keepdims=True)
    e = jnp.exp(logits - m)
    s = jnp.sum(e, axis=1, keepdims=True)
    p2 = e / s + jnp.float32(1e-8)
    ent_ref[...] = jnp.sum(-p2 * jnp.log(p2), axis=1, keepdims=True)


@jax.jit
def kernel(x, W, b):
    bp = b.reshape(1, K)
    grid = (B // BR,)
    idx, ent = pl.pallas_call(
        _fused_kernel,
        grid=grid,
        in_specs=[
            pl.BlockSpec((BR, D), lambda i: (i, 0)),
            pl.BlockSpec((K, D), lambda i: (0, 0)),
            pl.BlockSpec((1, K), lambda i: (0, 0)),
        ],
        out_specs=[
            pl.BlockSpec((BR, 1), lambda i: (i, 0)),
            pl.BlockSpec((BR, 1), lambda i: (i, 0)),
        ],
        out_shape=[
            jax.ShapeDtypeStruct((B, 1), jnp.int32),
            jax.ShapeDtypeStruct((B, 1), jnp.float32),
        ],
        compiler_params=pltpu.CompilerParams(
            dimension_semantics=("parallel",),
        ),
    )(x, W, bp)
    return idx, jnp.sum(ent)


# speedup vs baseline: 1.1585x; 1.0193x over previous
import jax
import jax.numpy as jnp
from jax.experimental import pallas as pl
from jax.experimental.pallas import tpu as pltpu

B = 4096
D = 2048
K = 1000
BR = 1024

_KS0 = 0
_KS1 = 42
_KS2 = 0x1BD11BDA ^ 42
_ROT_A = (13, 15, 26, 6)
_ROT_B = (17, 29, 16, 24)
_TINY = 1.1754943508222875e-38


def _rotl(x, d):
    return (x << jnp.uint32(d)) | (x >> jnp.uint32(32 - d))


def _four_rounds(x0, x1, rots):
    for r in rots:
        x0 = x0 + x1
        x1 = _rotl(x1, r) ^ x0
    return x0, x1


def _neg_log_neg_log_u(base, lin):
    x0 = jnp.full((BR, K), jnp.uint32(_KS0))
    x1 = lin + (base + jnp.uint32(_KS1))
    x0, x1 = _four_rounds(x0, x1, _ROT_A)
    x0, x1 = x0 + jnp.uint32(_KS1), x1 + jnp.uint32(_KS2 + 1)
    x0, x1 = _four_rounds(x0, x1, _ROT_B)
    x0, x1 = x0 + jnp.uint32(_KS2), x1 + jnp.uint32(_KS0 + 2)
    x0, x1 = _four_rounds(x0, x1, _ROT_A)
    x0, x1 = x0 + jnp.uint32(_KS0), x1 + jnp.uint32(_KS1 + 3)
    x0, x1 = _four_rounds(x0, x1, _ROT_B)
    x0, x1 = x0 + jnp.uint32(_KS1), x1 + jnp.uint32(_KS2 + 4)
    x0, x1 = _four_rounds(x0, x1, _ROT_A)
    x0, x1 = x0 + jnp.uint32(_KS2), x1 + jnp.uint32(_KS0 + 5)
    bits = x0 ^ x1
    fb = (bits >> jnp.uint32(9)) | jnp.uint32(0x3F800000)
    f = jax.lax.bitcast_convert_type(fb, jnp.float32) - jnp.float32(1.0)
    u = f + jnp.float32(_TINY)
    return jnp.log(-jnp.log(u))


def _fused_kernel(x_ref, w_ref, b_ref, idx_ref, ent_ref, lin_ref):
    i = pl.program_id(0)

    @pl.when(i == 0)
    def _make_lin():
        row = jax.lax.broadcasted_iota(jnp.uint32, (BR, K), 0)
        col = jax.lax.broadcasted_iota(jnp.uint32, (BR, K), 1)
        lin_ref[...] = row * jnp.uint32(K) + col

    logits = jax.lax.dot_general(
        x_ref[...], w_ref[...],
        dimension_numbers=(((1,), (1,)), ((), ())),
        preferred_element_type=jnp.float32) + b_ref[...]

    base = jnp.uint32(i * (BR * K))
    z = logits - _neg_log_neg_log_u(base, lin_ref[...])
    idx_ref[...] = jnp.argmax(z, axis=1).astype(jnp.int32)[:, None]

    m = jnp.max(logits, axis=1, keepdims=True)
    e = jnp.exp(logits - m)
    s = jnp.sum(e, axis=1, keepdims=True)
    p2 = e / s + jnp.float32(1e-8)
    ent_ref[...] = -jnp.sum(p2 * jnp.log(p2), axis=1, keepdims=True)


@jax.jit
def kernel(x, W, b):
    bp = b.reshape(1, K)
    grid = (B // BR,)
    idx, ent = pl.pallas_call(
        _fused_kernel,
        grid=grid,
        in_specs=[
            pl.BlockSpec((BR, D), lambda i: (i, 0)),
            pl.BlockSpec((K, D), lambda i: (0, 0)),
            pl.BlockSpec((1, K), lambda i: (0, 0)),
        ],
        out_specs=[
            pl.BlockSpec((BR, 1), lambda i: (i, 0)),
            pl.BlockSpec((BR, 1), lambda i: (i, 0)),
        ],
        out_shape=[
            jax.ShapeDtypeStruct((B, 1), jnp.int32),
            jax.ShapeDtypeStruct((B, 1), jnp.float32),
        ],
        scratch_shapes=[pltpu.VMEM((BR, K), jnp.uint32)],
        compiler_params=pltpu.CompilerParams(
            dimension_semantics=("arbitrary",),
        ),
    )(x, W, bp)
    return idx, jnp.sum(ent)


# speedup vs baseline: 1.2310x; 1.0626x over previous
import jax
import jax.numpy as jnp
from jax.experimental import pallas as pl
from jax.experimental.pallas import tpu as pltpu

B = 4096
D = 2048
K = 1000
BR = 1024

_KS0 = 0
_KS1 = 42
_KS2 = 0x1BD11BDA ^ 42
_ROT_A = (13, 15, 26, 6)
_ROT_B = (17, 29, 16, 24)
_TINY = 1.1754943508222875e-38


def _rotl(x, d):
    return (x << jnp.uint32(d)) | (x >> jnp.uint32(32 - d))


def _four_rounds(x0, x1, rots):
    for r in rots:
        x0 = x0 + x1
        x1 = _rotl(x1, r) ^ x0
    return x0, x1


def _neg_log_neg_log_u(base, lin):
    x0 = jnp.full((BR, K), jnp.uint32(_KS0))
    x1 = lin + (base + jnp.uint32(_KS1))
    x0, x1 = _four_rounds(x0, x1, _ROT_A)
    x0, x1 = x0 + jnp.uint32(_KS1), x1 + jnp.uint32(_KS2 + 1)
    x0, x1 = _four_rounds(x0, x1, _ROT_B)
    x0, x1 = x0 + jnp.uint32(_KS2), x1 + jnp.uint32(_KS0 + 2)
    x0, x1 = _four_rounds(x0, x1, _ROT_A)
    x0, x1 = x0 + jnp.uint32(_KS0), x1 + jnp.uint32(_KS1 + 3)
    x0, x1 = _four_rounds(x0, x1, _ROT_B)
    x0, x1 = x0 + jnp.uint32(_KS1), x1 + jnp.uint32(_KS2 + 4)
    x0, x1 = _four_rounds(x0, x1, _ROT_A)
    x0, x1 = x0 + jnp.uint32(_KS2), x1 + jnp.uint32(_KS0 + 5)
    bits = x0 ^ x1
    fb = (bits >> jnp.uint32(9)) | jnp.uint32(0x3F800000)
    f = jax.lax.bitcast_convert_type(fb, jnp.float32) - jnp.float32(1.0)
    u = f + jnp.float32(_TINY)
    return jnp.log(-jnp.log(u))


def _fused_kernel(x_ref, w_ref, b_ref, idx_ref, ent_ref, lin_ref):
    i = pl.program_id(0)

    @pl.when(i == 0)
    def _make_lin():
        row = jax.lax.broadcasted_iota(jnp.uint32, (BR, K), 0)
        col = jax.lax.broadcasted_iota(jnp.uint32, (BR, K), 1)
        lin_ref[...] = row * jnp.uint32(K) + col

    logits = jax.lax.dot_general(
        x_ref[...], w_ref[...],
        dimension_numbers=(((1,), (1,)), ((), ())),
        preferred_element_type=jnp.float32) + b_ref[...]

    base = jnp.uint32(i * (BR * K))
    z = logits - _neg_log_neg_log_u(base, lin_ref[...])
    idx_ref[...] = jnp.argmax(z, axis=1).astype(jnp.int32)[:, None]

    m = jnp.max(logits, axis=1, keepdims=True)
    e = jnp.exp(logits - m)
    s = jnp.sum(e, axis=1, keepdims=True)
    p2 = e / s + jnp.float32(1e-8)
    ent_sum = -jnp.sum(p2 * jnp.log(p2))

    @pl.when(i == 0)
    def _init():
        ent_ref[0, 0] = jnp.float32(0.0)

    ent_ref[0, 0] += ent_sum


@jax.jit
def kernel(x, W, b):
    bp = b.reshape(1, K)
    grid = (B // BR,)
    idx, ent = pl.pallas_call(
        _fused_kernel,
        grid=grid,
        in_specs=[
            pl.BlockSpec((BR, D), lambda i: (i, 0)),
            pl.BlockSpec((K, D), lambda i: (0, 0)),
            pl.BlockSpec((1, K), lambda i: (0, 0)),
        ],
        out_specs=[
            pl.BlockSpec((BR, 1), lambda i: (i, 0)),
            pl.BlockSpec(memory_space=pltpu.SMEM),
        ],
        out_shape=[
            jax.ShapeDtypeStruct((B, 1), jnp.int32),
            jax.ShapeDtypeStruct((1, 1), jnp.float32),
        ],
        scratch_shapes=[pltpu.VMEM((BR, K), jnp.uint32)],
        compiler_params=pltpu.CompilerParams(
            dimension_semantics=("arbitrary",),
        ),
    )(x, W, bp)
    return idx, ent[0, 0]


# speedup vs baseline: 1.2534x; 1.0182x over previous
import jax
import jax.numpy as jnp
from jax.experimental import pallas as pl
from jax.experimental.pallas import tpu as pltpu

B = 4096
D = 2048
K = 1000
BR = 1024

_KS0 = 0
_KS1 = 42
_KS2 = 0x1BD11BDA ^ 42
_ROT_A = (13, 15, 26, 6)
_ROT_B = (17, 29, 16, 24)
_TINY = 1.1754943508222875e-38


def _rotl(x, d):
    return (x << jnp.uint32(d)) | (x >> jnp.uint32(32 - d))


def _four_rounds(x0, x1, rots):
    for r in rots:
        x0 = x0 + x1
        x1 = _rotl(x1, r) ^ x0
    return x0, x1


def _neg_log_neg_log_u(base, lin):
    x0 = jnp.full((BR, K), jnp.uint32(_KS0))
    x1 = lin + (base + jnp.uint32(_KS1))
    x0, x1 = _four_rounds(x0, x1, _ROT_A)
    x0, x1 = x0 + jnp.uint32(_KS1), x1 + jnp.uint32(_KS2 + 1)
    x0, x1 = _four_rounds(x0, x1, _ROT_B)
    x0, x1 = x0 + jnp.uint32(_KS2), x1 + jnp.uint32(_KS0 + 2)
    x0, x1 = _four_rounds(x0, x1, _ROT_A)
    x0, x1 = x0 + jnp.uint32(_KS0), x1 + jnp.uint32(_KS1 + 3)
    x0, x1 = _four_rounds(x0, x1, _ROT_B)
    x0, x1 = x0 + jnp.uint32(_KS1), x1 + jnp.uint32(_KS2 + 4)
    x0, x1 = _four_rounds(x0, x1, _ROT_A)
    x0, x1 = x0 + jnp.uint32(_KS2), x1 + jnp.uint32(_KS0 + 5)
    bits = x0 ^ x1
    fb = (bits >> jnp.uint32(9)) | jnp.uint32(0x3F800000)
    f = jax.lax.bitcast_convert_type(fb, jnp.float32) - jnp.float32(1.0)
    u = f + jnp.float32(_TINY)
    return jnp.log(-jnp.log(u))


def _fused_kernel(x_ref, w_ref, b_ref, idx_ref, ent_ref, lin_ref):
    i = pl.program_id(0)

    @pl.when(i == 0)
    def _make_lin():
        row = jax.lax.broadcasted_iota(jnp.uint32, (BR, K), 0)
        col = jax.lax.broadcasted_iota(jnp.uint32, (BR, K), 1)
        lin_ref[...] = row * jnp.uint32(K) + col

    logits = jax.lax.dot_general(
        x_ref[...], w_ref[...],
        dimension_numbers=(((1,), (1,)), ((), ())),
        preferred_element_type=jnp.float32) + b_ref[...]

    base = jnp.uint32(i * (BR * K))
    z = logits - _neg_log_neg_log_u(base, lin_ref[...])
    idx_ref[...] = jnp.argmax(z, axis=1).astype(jnp.int32)[:, None]

    e = jnp.exp(logits)
    s = jnp.sum(e, axis=1, keepdims=True)
    p2 = e / s + jnp.float32(1e-8)
    ent_sum = -jnp.sum(p2 * jnp.log(p2))

    @pl.when(i == 0)
    def _init():
        ent_ref[0, 0] = jnp.float32(0.0)

    ent_ref[0, 0] += ent_sum


@jax.jit
def kernel(x, W, b):
    bp = b.reshape(1, K)
    grid = (B // BR,)
    idx, ent = pl.pallas_call(
        _fused_kernel,
        grid=grid,
        in_specs=[
            pl.BlockSpec((BR, D), lambda i: (i, 0)),
            pl.BlockSpec((K, D), lambda i: (0, 0)),
            pl.BlockSpec((1, K), lambda i: (0, 0)),
        ],
        out_specs=[
            pl.BlockSpec((BR, 1), lambda i: (i, 0)),
            pl.BlockSpec(memory_space=pltpu.SMEM),
        ],
        out_shape=[
            jax.ShapeDtypeStruct((B, 1), jnp.int32),
            jax.ShapeDtypeStruct((1, 1), jnp.float32),
        ],
        scratch_shapes=[pltpu.VMEM((BR, K), jnp.uint32)],
        compiler_params=pltpu.CompilerParams(
            dimension_semantics=("arbitrary",),
        ),
    )(x, W, bp)
    return idx, ent[0, 0]
